# baseline (speedup 1.0000x reference)
import functools

import jax
import jax.numpy as jnp
from jax import lax
from jax.experimental import pallas as pl
from jax.experimental.pallas import tpu as pltpu
from jax.experimental.pallas import tpu_sc as plsc

_NC = 2
_NS = 16
_NW = _NC * _NS

_BM = 128


def _sc_scatter_rows(x, rank, n, d):
    b_per_w = n // _NW
    mesh = plsc.VectorSubcoreMesh(core_axis_name="c", subcore_axis_name="s")

    @functools.partial(
        pl.kernel,
        mesh=mesh,
        out_type=jax.ShapeDtypeStruct((n, d), jnp.float32),
        scratch_types=[
            pltpu.VMEM((b_per_w,), jnp.int32),
            pltpu.VMEM((b_per_w, d), jnp.float32),
            pltpu.SemaphoreType.DMA,
        ],
    )
    def scatter_kernel(x_hbm, rank_hbm, out_hbm, idx_v, rows_v, sem):
        wid = lax.axis_index("s") * _NC + lax.axis_index("c")
        base = wid * b_per_w
        pltpu.sync_copy(rank_hbm.at[pl.ds(base, b_per_w)], idx_v)
        pltpu.sync_copy(x_hbm.at[pl.ds(base, b_per_w)], rows_v)
        pltpu.async_copy(rows_v, out_hbm.at[idx_v], sem).wait()

    return scatter_kernel(x, rank)


def _gmm_kernel(gid_ref, blk_ref, lo_ref, hi_ref, x_ref, w_ref, b_ref, o_ref):
    s = pl.program_id(0)
    lo = lo_ref[s]
    hi = hi_ref[s]
    blk = blk_ref[s]
    bm = o_ref.shape[0]
    rows = blk * bm + lax.broadcasted_iota(jnp.int32, (bm, 1), 0)
    mask = (rows >= lo) & (rows < hi)
    acc = jnp.dot(x_ref[...], w_ref[0], preferred_element_type=jnp.float32)
    y = jnp.maximum(acc + b_ref[0], 0.0)
    o_ref[...] = jnp.where(mask, y, o_ref[...])


def _grouped_matmul(x_sorted, W, b, gid, blk, lo, hi, maxp):
    n, din = x_sorted.shape
    e, _, dout = W.shape
    grid_spec = pltpu.PrefetchScalarGridSpec(
        num_scalar_prefetch=4,
        grid=(maxp,),
        in_specs=[
            pl.BlockSpec((_BM, din), lambda s, g, bk, l, h: (bk[s], 0)),
            pl.BlockSpec((1, din, dout), lambda s, g, bk, l, h: (g[s], 0, 0)),
            pl.BlockSpec((1, 1, dout), lambda s, g, bk, l, h: (g[s], 0, 0)),
        ],
        out_specs=pl.BlockSpec((_BM, dout), lambda s, g, bk, l, h: (bk[s], 0)),
    )
    return pl.pallas_call(
        _gmm_kernel,
        grid_spec=grid_spec,
        out_shape=jax.ShapeDtypeStruct((n, dout), jnp.float32),
    )(gid, blk, lo, hi, x_sorted, W, b.reshape(e, 1, dout))


def _routing(flat, n, e, nb, bm):
    maxp = nb + e - 1
    erange = jnp.arange(e, dtype=jnp.int32)
    oh = (flat[:, None] == erange[None, :]).astype(jnp.int32)
    cum = jnp.cumsum(oh, axis=0)
    counts = cum[-1]
    offsets = jnp.cumsum(counts) - counts
    occ = jnp.take_along_axis(cum, flat[:, None], axis=1)[:, 0] - 1
    rank = jnp.take(offsets, flat) + occ

    first = offsets // bm
    last = (offsets + counts - 1) // bm
    t = jnp.where(counts > 0, last - first + 1, 0)
    tcum = jnp.cumsum(t)
    p_total = tcum[-1]
    parange = jnp.arange(maxp, dtype=jnp.int32)
    le = (tcum[None, :] <= parange[:, None]).astype(jnp.int32)
    eidx = jnp.sum(le, axis=1)
    e_pad = jnp.sum((tcum <= p_total - 1).astype(jnp.int32))
    valid = parange < p_total
    eidx = jnp.where(valid, eidx, e_pad)
    k = parange - (tcum[eidx] - t[eidx])
    blk = jnp.minimum(first[eidx] + k, nb - 1)
    lo = jnp.where(valid, offsets[eidx], 0)
    hi = jnp.where(valid, offsets[eidx] + counts[eidx], 0)
    return rank, eidx, blk, lo, hi, maxp


def kernel(input_batch, indices, W, b):
    n, d = input_batch.shape
    e = W.shape[0]
    nb = n // _BM
    flat = indices[:, 0].astype(jnp.int32)
    rank, gid, blk, lo, hi, maxp = _routing(flat, n, e, nb, _BM)
    x_sorted = _sc_scatter_rows(input_batch, rank, n, d)
    return _grouped_matmul(x_sorted, W, b, gid, blk, lo, hi, maxp)


# speedup vs baseline: 1.5183x; 1.5183x over previous
import functools

import jax
import jax.numpy as jnp
from jax import lax
from jax.experimental import pallas as pl
from jax.experimental.pallas import tpu as pltpu
from jax.experimental.pallas import tpu_sc as plsc

_NC = 2
_NS = 16
_NW = _NC * _NS

_BM = 128


def _sc_scatter_rows(x, rank, n, d):
    b_per_w = n // _NW
    mesh = plsc.VectorSubcoreMesh(core_axis_name="c", subcore_axis_name="s")

    @functools.partial(
        pl.kernel,
        mesh=mesh,
        out_type=jax.ShapeDtypeStruct((n, d), jnp.float32),
        scratch_types=[
            pltpu.VMEM((b_per_w,), jnp.int32),
            pltpu.VMEM((b_per_w, d), jnp.float32),
            pltpu.SemaphoreType.DMA,
        ],
    )
    def scatter_kernel(x_hbm, rank_hbm, out_hbm, idx_v, rows_v, sem):
        wid = lax.axis_index("s") * _NC + lax.axis_index("c")
        base = wid * b_per_w
        pltpu.sync_copy(rank_hbm.at[pl.ds(base, b_per_w)], idx_v)
        pltpu.sync_copy(x_hbm.at[pl.ds(base, b_per_w)], rows_v)
        pltpu.async_copy(rows_v, out_hbm.at[idx_v], sem).wait()

    return scatter_kernel(x, rank)


def _gmm_kernel(gid_ref, blk_ref, lo_ref, hi_ref, x_ref, w_ref, b_ref, o_ref):
    s = pl.program_id(0)
    lo = lo_ref[s]
    hi = hi_ref[s]
    blk = blk_ref[s]
    bm = o_ref.shape[0]
    rows = blk * bm + lax.broadcasted_iota(jnp.int32, (bm, 1), 0)
    mask = (rows >= lo) & (rows < hi)
    acc = jnp.dot(x_ref[...], w_ref[0], preferred_element_type=jnp.float32)
    y = jnp.maximum(acc + b_ref[0], 0.0)
    o_ref[...] = jnp.where(mask, y, o_ref[...])


def _grouped_matmul(x_sorted, W, b, gid, blk, lo, hi, maxp):
    n, din = x_sorted.shape
    e, _, dout = W.shape
    grid_spec = pltpu.PrefetchScalarGridSpec(
        num_scalar_prefetch=4,
        grid=(maxp,),
        in_specs=[
            pl.BlockSpec((_BM, din), lambda s, g, bk, l, h: (bk[s], 0)),
            pl.BlockSpec((1, din, dout), lambda s, g, bk, l, h: (g[s], 0, 0)),
            pl.BlockSpec((1, 1, dout), lambda s, g, bk, l, h: (g[s], 0, 0)),
        ],
        out_specs=pl.BlockSpec((_BM, dout), lambda s, g, bk, l, h: (bk[s], 0)),
    )
    return pl.pallas_call(
        _gmm_kernel,
        grid_spec=grid_spec,
        out_shape=jax.ShapeDtypeStruct((n, dout), jnp.float32),
    )(gid, blk, lo, hi, x_sorted, W, b.reshape(e, 1, dout))


def _routing(flat, n, e, nb, bm):
    maxp = nb + e - 1
    erange = jnp.arange(e, dtype=jnp.int32)
    irange = jnp.arange(n, dtype=jnp.int32)
    fi = flat[:, None]
    fj = flat[None, :]
    before = (fj < fi) | ((fj == fi) & (irange[None, :] < irange[:, None]))
    rank = jnp.sum(before.astype(jnp.int32), axis=1)
    counts = jnp.sum((flat[None, :] == erange[:, None]).astype(jnp.int32), axis=1)
    offsets = jnp.cumsum(counts) - counts

    first = offsets // bm
    last = (offsets + counts - 1) // bm
    t = jnp.where(counts > 0, last - first + 1, 0)
    tcum = jnp.cumsum(t)
    p_total = tcum[-1]
    parange = jnp.arange(maxp, dtype=jnp.int32)
    le = (tcum[None, :] <= parange[:, None]).astype(jnp.int32)
    eidx = jnp.sum(le, axis=1)
    e_pad = jnp.sum((tcum <= p_total - 1).astype(jnp.int32))
    valid = parange < p_total
    eidx = jnp.where(valid, eidx, e_pad)
    k = parange - (tcum[eidx] - t[eidx])
    blk = jnp.minimum(first[eidx] + k, nb - 1)
    lo = jnp.where(valid, offsets[eidx], 0)
    hi = jnp.where(valid, offsets[eidx] + counts[eidx], 0)
    return rank, eidx, blk, lo, hi, maxp


def kernel(input_batch, indices, W, b):
    n, d = input_batch.shape
    e = W.shape[0]
    nb = n // _BM
    flat = indices[:, 0].astype(jnp.int32)
    rank, gid, blk, lo, hi, maxp = _routing(flat, n, e, nb, _BM)
    x_sorted = _sc_scatter_rows(input_batch, rank, n, d)
    return _grouped_matmul(x_sorted, W, b, gid, blk, lo, hi, maxp)


# speedup vs baseline: 1.5978x; 1.0524x over previous
import functools

import jax
import jax.numpy as jnp
from jax import lax
from jax.experimental import pallas as pl
from jax.experimental.pallas import tpu as pltpu
from jax.experimental.pallas import tpu_sc as plsc

_NC = 2
_NS = 16
_NW = _NC * _NS

_BM = 128
_MP = 128


def _routing_kernel(idx_ref, rank_ref, maps_ref, gcum_ref):
    n = idx_ref.shape[0]
    e = gcum_ref.shape[1]
    nb = n // _BM
    nchunks = n // 128
    flat_col = idx_ref[...]
    erange_row = lax.broadcasted_iota(jnp.int32, (1, e), 1)

    ir = lax.broadcasted_iota(jnp.int32, (128, 128), 0)
    ic = lax.broadcasted_iota(jnp.int32, (128, 128), 1)
    ltri = (ir >= ic).astype(jnp.float32)
    h = jnp.zeros((1, e), jnp.int32)
    for r in range(nchunks):
        fc = flat_col[r * 128:(r + 1) * 128]
        oh = (fc == erange_row).astype(jnp.float32)
        cum = jnp.dot(ltri, oh, preferred_element_type=jnp.float32)
        gcum_ref[r * 128:(r + 1) * 128, :] = cum.astype(jnp.int32) + h
        h = h + jnp.sum(oh.astype(jnp.int32), axis=0, keepdims=True)
    counts_row = h

    er_s = lax.broadcasted_iota(jnp.int32, (e, e), 0)
    er_l = lax.broadcasted_iota(jnp.int32, (e, e), 1)
    diag = (er_s == er_l).astype(jnp.int32)
    counts_col = jnp.sum(diag * counts_row, axis=1, keepdims=True)
    offsets_row = jnp.sum((er_s < er_l).astype(jnp.int32) * counts_col,
                          axis=0, keepdims=True)
    offsets_col = jnp.sum((er_l < er_s).astype(jnp.int32) * counts_row,
                          axis=1, keepdims=True)

    add_row = offsets_row - 1
    for r in range(nchunks):
        fc = flat_col[r * 128:(r + 1) * 128]
        ohi = (fc == erange_row).astype(jnp.int32)
        a = gcum_ref[r * 128:(r + 1) * 128, :] + add_row
        rank_ref[r * 128:(r + 1) * 128, :] = jnp.sum(ohi * a, axis=1,
                                                     keepdims=True)

    first_col = offsets_col // _BM
    last_col = (offsets_col + counts_col - 1) // _BM
    t_col = jnp.where(counts_col > 0, last_col - first_col + 1, 0)
    t_row = jnp.sum(diag * t_col, axis=0, keepdims=True)
    tcum_col = jnp.sum((er_l <= er_s).astype(jnp.int32) * t_row,
                       axis=1, keepdims=True)
    p_total = jnp.sum(t_col, keepdims=True)
    p_row = lax.broadcasted_iota(jnp.int32, (1, _MP), 1)
    le = (tcum_col <= p_row).astype(jnp.int32)
    eidx_row = jnp.sum(le, axis=0, keepdims=True)
    e_pad = jnp.sum((tcum_col <= p_total - 1).astype(jnp.int32),
                    axis=0, keepdims=True)
    valid = p_row < p_total
    eidx_row = jnp.where(valid, eidx_row, e_pad)
    erange_col = lax.broadcasted_iota(jnp.int32, (e, 1), 0)
    ohp = (erange_col == eidx_row).astype(jnp.int32)

    def gat(vec_col):
        return jnp.sum(ohp * vec_col, axis=0, keepdims=True)

    t_g = gat(t_col)
    tcum_g = gat(tcum_col)
    first_g = gat(first_col)
    off_g = gat(offsets_col)
    cnt_g = gat(counts_col)
    k = p_row - (tcum_g - t_g)
    blk = jnp.minimum(first_g + k, nb - 1)
    maps_ref[0:1, :] = eidx_row
    maps_ref[1:2, :] = blk
    maps_ref[2:3, :] = jnp.where(valid, off_g, 0)
    maps_ref[3:4, :] = jnp.where(valid, off_g + cnt_g, 0)


def _routing(indices, n, e):
    rank2d, maps = pl.pallas_call(
        _routing_kernel,
        out_shape=(
            jax.ShapeDtypeStruct((n, 1), jnp.int32),
            jax.ShapeDtypeStruct((4, _MP), jnp.int32),
        ),
        scratch_shapes=[pltpu.VMEM((n, e), jnp.int32)],
    )(indices)
    return rank2d.reshape(n), maps


def _sc_scatter_rows(x, rank, n, d):
    b_per_w = n // _NW
    mesh = plsc.VectorSubcoreMesh(core_axis_name="c", subcore_axis_name="s")

    @functools.partial(
        pl.kernel,
        mesh=mesh,
        out_type=jax.ShapeDtypeStruct((n, d), jnp.float32),
        scratch_types=[
            pltpu.VMEM((b_per_w,), jnp.int32),
            pltpu.VMEM((b_per_w, d), jnp.float32),
            pltpu.SemaphoreType.DMA,
        ],
    )
    def scatter_kernel(x_hbm, rank_hbm, out_hbm, idx_v, rows_v, sem):
        wid = lax.axis_index("s") * _NC + lax.axis_index("c")
        base = wid * b_per_w
        pltpu.sync_copy(rank_hbm.at[pl.ds(base, b_per_w)], idx_v)
        pltpu.sync_copy(x_hbm.at[pl.ds(base, b_per_w)], rows_v)
        pltpu.async_copy(rows_v, out_hbm.at[idx_v], sem).wait()

    return scatter_kernel(x, rank)


def _gmm_kernel(m_ref, x_ref, w_ref, b_ref, o_ref):
    s = pl.program_id(0)
    lo = m_ref[2, s]
    hi = m_ref[3, s]
    blk = m_ref[1, s]
    bm = o_ref.shape[0]
    rows = blk * bm + lax.broadcasted_iota(jnp.int32, (bm, 1), 0)
    mask = (rows >= lo) & (rows < hi)
    acc = jnp.dot(x_ref[...], w_ref[0], preferred_element_type=jnp.float32)
    y = jnp.maximum(acc + b_ref[0], 0.0)
    o_ref[...] = jnp.where(mask, y, o_ref[...])


def _grouped_matmul(x_sorted, W, b, maps, maxp):
    n, din = x_sorted.shape
    e, _, dout = W.shape
    grid_spec = pltpu.PrefetchScalarGridSpec(
        num_scalar_prefetch=1,
        grid=(maxp,),
        in_specs=[
            pl.BlockSpec((_BM, din), lambda s, m: (m[1, s], 0)),
            pl.BlockSpec((1, din, dout), lambda s, m: (m[0, s], 0, 0)),
            pl.BlockSpec((1, 1, dout), lambda s, m: (m[0, s], 0, 0)),
        ],
        out_specs=pl.BlockSpec((_BM, dout), lambda s, m: (m[1, s], 0)),
    )
    return pl.pallas_call(
        _gmm_kernel,
        grid_spec=grid_spec,
        out_shape=jax.ShapeDtypeStruct((n, dout), jnp.float32),
    )(maps, x_sorted, W, b.reshape(e, 1, dout))


def kernel(input_batch, indices, W, b):
    n, d = input_batch.shape
    e = W.shape[0]
    nb = n // _BM
    maxp = nb + e - 1
    rank, maps = _routing(indices.astype(jnp.int32), n, e)
    x_sorted = _sc_scatter_rows(input_batch, rank, n, d)
    return _grouped_matmul(x_sorted, W, b, maps, maxp)
